# Initial kernel scaffold; baseline (speedup 1.0000x reference)
import jax
import jax.numpy as jnp
from jax.experimental import pallas as pl


def kernel(x, adj, weight, a, bias):
    raise NotImplementedError("write your pallas kernel here")


# speedup vs baseline: 4.8521x; 4.8521x over previous
import functools

import jax
import jax.numpy as jnp
from jax import lax
from jax.experimental import pallas as pl
from jax.experimental.pallas import tpu as pltpu
from jax.experimental.pallas import tpu_sc as plsc

N = 10000
D = 128
O = 128
E = 160000
ALPHA = 0.2

NC = 2
NS = 16
LANES = 16
EPT16 = E // NS
FPS = 8
NH = N // 2
ECH = 2048
NCHUNK = E // ECH
ETAIL = E - NCHUNK * ECH
BROWS = 1000


def _prologue_body(x_ref, w_ref, a1_ref, a2_ref, h_ref, wh1_ref,
                   wh2_ref, s_ref):
    i = pl.program_id(0)
    xb = x_ref[...]
    hb = jnp.dot(xb, w_ref[...], preferred_element_type=jnp.float32)
    h_ref[...] = hb
    wh1_ref[...] = jnp.sum(hb * a1_ref[...], axis=1, keepdims=True)
    wh2_ref[...] = jnp.sum(hb * a2_ref[...], axis=1, keepdims=True)

    @pl.when(i == 0)
    def _():
        s_ref[...] = jnp.zeros_like(s_ref)

    s_ref[...] += jnp.sum(hb, axis=0, keepdims=True)


@functools.cache
def _prologue():
    return pl.pallas_call(
        _prologue_body,
        grid=(N // BROWS,),
        in_specs=[
            pl.BlockSpec((BROWS, D), lambda i: (i, 0)),
            pl.BlockSpec((D, O), lambda i: (0, 0)),
            pl.BlockSpec((1, O), lambda i: (0, 0)),
            pl.BlockSpec((1, O), lambda i: (0, 0)),
        ],
        out_specs=[
            pl.BlockSpec((BROWS, O), lambda i: (i, 0)),
            pl.BlockSpec((BROWS, 1), lambda i: (i, 0)),
            pl.BlockSpec((BROWS, 1), lambda i: (i, 0)),
            pl.BlockSpec((1, O), lambda i: (0, 0)),
        ],
        out_shape=[
            jax.ShapeDtypeStruct((N, O), jnp.float32),
            jax.ShapeDtypeStruct((N, 1), jnp.float32),
            jax.ShapeDtypeStruct((N, 1), jnp.float32),
            jax.ShapeDtypeStruct((1, O), jnp.float32),
        ],
    )


def _edge_kernel_body(src_hbm, dst_hbm, wh1_hbm, wh2_hbm,
                      w_hbm, zpart_hbm,
                      srcv, dstv, wh1v, wh2v, wv):
    cid = lax.axis_index("c")
    sid = lax.axis_index("s")
    base = sid * EPT16

    pltpu.sync_copy(wh1_hbm, wh1v)
    pltpu.sync_copy(wh2_hbm, wh2v)
    pltpu.sync_copy(src_hbm.at[pl.ds(base, EPT16)], srcv)
    pltpu.sync_copy(dst_hbm.at[pl.ds(base, EPT16)], dstv)

    @pl.when(cid != 0)
    def _w_path():

        @pl.loop(0, EPT16, step=LANES)
        def _(j):
            s = srcv[pl.ds(j, LANES)]
            d = dstv[pl.ds(j, LANES)]
            v = plsc.load_gather(wh1v, [s]) + plsc.load_gather(wh2v, [d])
            v = jnp.maximum(v, ALPHA * v)
            wv[pl.ds(j, LANES)] = jnp.exp(v) - 1.0

        pltpu.sync_copy(wv, w_hbm.at[pl.ds(base, EPT16)])

    @pl.when(cid == 0)
    def _z_path():
        @pl.loop(0, N, step=LANES)
        def _(j):
            wv[pl.ds(j, LANES)] = jnp.zeros((LANES,), jnp.float32)

        @pl.loop(0, EPT16, step=LANES)
        def _(j):
            s = srcv[pl.ds(j, LANES)]
            d = dstv[pl.ds(j, LANES)]
            v = plsc.load_gather(wh1v, [s]) + plsc.load_gather(wh2v, [d])
            v = jnp.maximum(v, ALPHA * v)
            plsc.addupdate_scatter(wv, [s], jnp.exp(v) - 1.0)

        pltpu.sync_copy(wv, zpart_hbm.at[sid])


@functools.cache
def _edge_kernel():
    mesh = plsc.VectorSubcoreMesh(
        core_axis_name="c", subcore_axis_name="s",
        num_cores=NC, num_subcores=NS)
    return pl.kernel(
        _edge_kernel_body,
        out_type=(
            jax.ShapeDtypeStruct((E,), jnp.float32),
            jax.ShapeDtypeStruct((NS, N), jnp.float32),
        ),
        mesh=mesh,
        compiler_params=pltpu.CompilerParams(needs_layout_passes=False),
        scratch_types=[
            pltpu.VMEM((EPT16,), jnp.int32),
            pltpu.VMEM((EPT16,), jnp.int32),
            pltpu.VMEM((N,), jnp.float32),
            pltpu.VMEM((N,), jnp.float32),
            pltpu.VMEM((EPT16,), jnp.float32),
        ],
    )


def _agg_kernel_body(ht_hbm, src_hbm, dst_hbm, w_hbm,
                     pt_out,
                     slab, acc, se, de, we):
    cid = lax.axis_index("c")
    sid = lax.axis_index("s")
    g = cid * NS + sid
    m = g % NS
    q = g // NS
    sb = q * NH

    pltpu.sync_copy(ht_hbm.at[pl.ds(m * FPS, FPS)], slab)

    for f in range(FPS):
        @pl.loop(0, NH, step=LANES)
        def _(j):
            acc[f, pl.ds(j, LANES)] = jnp.zeros((LANES,), jnp.float32)

    def _scan_chunk(off, size):
        pltpu.sync_copy(src_hbm.at[pl.ds(off, size)], se.at[pl.ds(0, size)])
        pltpu.sync_copy(dst_hbm.at[pl.ds(off, size)], de.at[pl.ds(0, size)])
        pltpu.sync_copy(w_hbm.at[pl.ds(off, size)], we.at[pl.ds(0, size)])

        @pl.loop(0, size, step=LANES)
        def _(j):
            s = se[pl.ds(j, LANES)]
            d = de[pl.ds(j, LANES)]
            wv = we[pl.ds(j, LANES)]
            sloc = s - sb
            mask = (s >= sb) & (sloc < NH)
            for f in range(FPS):
                frow = jnp.zeros((LANES,), jnp.int32) + f
                hval = plsc.load_gather(slab, [frow, d])
                plsc.addupdate_scatter(acc, [frow, sloc], hval * wv,
                                       mask=mask)

    @pl.loop(0, NCHUNK)
    def _(c):
        _scan_chunk(c * ECH, ECH)

    _scan_chunk(NCHUNK * ECH, ETAIL)

    pltpu.sync_copy(acc, pt_out.at[q, pl.ds(m * FPS, FPS)])


@functools.cache
def _agg_kernel():
    mesh = plsc.VectorSubcoreMesh(
        core_axis_name="c", subcore_axis_name="s",
        num_cores=NC, num_subcores=NS)
    return pl.kernel(
        _agg_kernel_body,
        out_type=jax.ShapeDtypeStruct((2, O, NH), jnp.float32),
        mesh=mesh,
        compiler_params=pltpu.CompilerParams(needs_layout_passes=False),
        scratch_types=[
            pltpu.VMEM((FPS, N), jnp.float32),
            pltpu.VMEM((FPS, NH), jnp.float32),
            pltpu.VMEM((ECH,), jnp.int32),
            pltpu.VMEM((ECH,), jnp.int32),
            pltpu.VMEM((ECH,), jnp.float32),
        ],
    )


def _epilogue_body(p_ref, zp_ref, s_ref, b_ref, o_ref):
    p = p_ref[...]
    z = jnp.sum(zp_ref[...], axis=1, keepdims=True) + jnp.float32(N)
    t = (p + s_ref[...]) / z
    t = jnp.maximum(t, ALPHA * t)
    nrm = jnp.maximum(jnp.sqrt(jnp.sum(t * t, axis=1, keepdims=True)),
                      jnp.float32(1e-12))
    o_ref[...] = t / nrm + b_ref[...]


@functools.cache
def _epilogue():
    return pl.pallas_call(
        _epilogue_body,
        grid=(N // BROWS,),
        in_specs=[
            pl.BlockSpec((BROWS, O), lambda i: (i, 0)),
            pl.BlockSpec((BROWS, NS), lambda i: (i, 0)),
            pl.BlockSpec((1, O), lambda i: (0, 0)),
            pl.BlockSpec((1, O), lambda i: (0, 0)),
        ],
        out_specs=pl.BlockSpec((BROWS, O), lambda i: (i, 0)),
        out_shape=jax.ShapeDtypeStruct((N, O), jnp.float32),
    )


def kernel(x, adj, weight, a, bias):
    a1 = a[:O, 0].reshape(1, O)
    a2 = a[O:, 0].reshape(1, O)
    h, wh1, wh2, s = _prologue()(x, weight, a1, a2)
    src = adj[0]
    dst = adj[1]
    w, zpart = _edge_kernel()(src, dst, wh1.reshape(N), wh2.reshape(N))
    ht = h.T
    pt = _agg_kernel()(ht, src, dst, w)
    p = jnp.concatenate([pt[0].T, pt[1].T], axis=0)
    return _epilogue()(p, zpart.T, s, bias.reshape(1, O))


# speedup vs baseline: 5.1149x; 1.0542x over previous
import functools

import jax
import jax.numpy as jnp
from jax import lax
from jax.experimental import pallas as pl
from jax.experimental.pallas import tpu as pltpu
from jax.experimental.pallas import tpu_sc as plsc

N = 10000
D = 128
O = 128
E = 160000
ALPHA = 0.2

NC = 2
NS = 16
LANES = 16
EPT16 = E // NS
FPS = 8
NH = N // 2
ECH = 3072
NCHUNK = E // ECH
ETAIL = E - NCHUNK * ECH
BROWS = 1000


def _prologue_body(x_ref, w_ref, a1_ref, a2_ref, h_ref, wh1_ref,
                   wh2_ref, s_ref):
    i = pl.program_id(0)
    xb = x_ref[...]
    hb = jnp.dot(xb, w_ref[...], preferred_element_type=jnp.float32)
    h_ref[...] = hb
    wh1_ref[...] = jnp.sum(hb * a1_ref[...], axis=1, keepdims=True)
    wh2_ref[...] = jnp.sum(hb * a2_ref[...], axis=1, keepdims=True)

    @pl.when(i == 0)
    def _():
        s_ref[...] = jnp.zeros_like(s_ref)

    s_ref[...] += jnp.sum(hb, axis=0, keepdims=True)


@functools.cache
def _prologue():
    return pl.pallas_call(
        _prologue_body,
        grid=(N // BROWS,),
        in_specs=[
            pl.BlockSpec((BROWS, D), lambda i: (i, 0)),
            pl.BlockSpec((D, O), lambda i: (0, 0)),
            pl.BlockSpec((1, O), lambda i: (0, 0)),
            pl.BlockSpec((1, O), lambda i: (0, 0)),
        ],
        out_specs=[
            pl.BlockSpec((BROWS, O), lambda i: (i, 0)),
            pl.BlockSpec((BROWS, 1), lambda i: (i, 0)),
            pl.BlockSpec((BROWS, 1), lambda i: (i, 0)),
            pl.BlockSpec((1, O), lambda i: (0, 0)),
        ],
        out_shape=[
            jax.ShapeDtypeStruct((N, O), jnp.float32),
            jax.ShapeDtypeStruct((N, 1), jnp.float32),
            jax.ShapeDtypeStruct((N, 1), jnp.float32),
            jax.ShapeDtypeStruct((1, O), jnp.float32),
        ],
    )


def _edge_kernel_body(src_hbm, dst_hbm, wh1_hbm, wh2_hbm,
                      w_hbm, zpart_hbm,
                      srcv, dstv, wh1v, wh2v, wv):
    cid = lax.axis_index("c")
    sid = lax.axis_index("s")
    base = sid * EPT16

    pltpu.sync_copy(wh1_hbm, wh1v)
    pltpu.sync_copy(wh2_hbm, wh2v)
    pltpu.sync_copy(src_hbm.at[pl.ds(base, EPT16)], srcv)
    pltpu.sync_copy(dst_hbm.at[pl.ds(base, EPT16)], dstv)

    @pl.when(cid != 0)
    def _w_path():

        @pl.loop(0, EPT16, step=LANES)
        def _(j):
            s = srcv[pl.ds(j, LANES)]
            d = dstv[pl.ds(j, LANES)]
            v = plsc.load_gather(wh1v, [s]) + plsc.load_gather(wh2v, [d])
            v = jnp.maximum(v, ALPHA * v)
            wv[pl.ds(j, LANES)] = jnp.exp(v) - 1.0

        pltpu.sync_copy(wv, w_hbm.at[pl.ds(base, EPT16)])

    @pl.when(cid == 0)
    def _z_path():
        @pl.loop(0, N, step=LANES)
        def _(j):
            wv[pl.ds(j, LANES)] = jnp.zeros((LANES,), jnp.float32)

        @pl.loop(0, EPT16, step=LANES)
        def _(j):
            s = srcv[pl.ds(j, LANES)]
            d = dstv[pl.ds(j, LANES)]
            v = plsc.load_gather(wh1v, [s]) + plsc.load_gather(wh2v, [d])
            v = jnp.maximum(v, ALPHA * v)
            plsc.addupdate_scatter(wv, [s], jnp.exp(v) - 1.0)

        pltpu.sync_copy(wv, zpart_hbm.at[sid])


@functools.cache
def _edge_kernel():
    mesh = plsc.VectorSubcoreMesh(
        core_axis_name="c", subcore_axis_name="s",
        num_cores=NC, num_subcores=NS)
    return pl.kernel(
        _edge_kernel_body,
        out_type=(
            jax.ShapeDtypeStruct((E,), jnp.float32),
            jax.ShapeDtypeStruct((NS, N), jnp.float32),
        ),
        mesh=mesh,
        compiler_params=pltpu.CompilerParams(needs_layout_passes=False),
        scratch_types=[
            pltpu.VMEM((EPT16,), jnp.int32),
            pltpu.VMEM((EPT16,), jnp.int32),
            pltpu.VMEM((N,), jnp.float32),
            pltpu.VMEM((N,), jnp.float32),
            pltpu.VMEM((EPT16,), jnp.float32),
        ],
    )


def _agg_kernel_body(ht_hbm, src_hbm, dst_hbm, w_hbm,
                     pt_out,
                     slab, acc, se, de, we):
    cid = lax.axis_index("c")
    sid = lax.axis_index("s")
    g = cid * NS + sid
    m = g % NS
    q = g // NS
    sb = q * NH

    pltpu.sync_copy(ht_hbm.at[pl.ds(m * FPS, FPS)], slab)

    for f in range(FPS):
        @pl.loop(0, NH, step=LANES)
        def _(j):
            acc[f, pl.ds(j, LANES)] = jnp.zeros((LANES,), jnp.float32)

    def _scan_chunk(off, size):
        pltpu.sync_copy(src_hbm.at[pl.ds(off, size)], se.at[pl.ds(0, size)])
        pltpu.sync_copy(dst_hbm.at[pl.ds(off, size)], de.at[pl.ds(0, size)])
        pltpu.sync_copy(w_hbm.at[pl.ds(off, size)], we.at[pl.ds(0, size)])

        @pl.loop(0, size, step=LANES)
        def _(j):
            s = se[pl.ds(j, LANES)]
            d = de[pl.ds(j, LANES)]
            wv = we[pl.ds(j, LANES)]
            sloc = s - sb
            mask = (s >= sb) & (sloc < NH)
            for f in range(FPS):
                frow = jnp.zeros((LANES,), jnp.int32) + f
                hval = plsc.load_gather(slab, [frow, d])
                plsc.addupdate_scatter(acc, [frow, sloc], hval * wv,
                                       mask=mask)

    @pl.loop(0, NCHUNK)
    def _(c):
        _scan_chunk(c * ECH, ECH)

    _scan_chunk(NCHUNK * ECH, ETAIL)

    pltpu.sync_copy(acc, pt_out.at[q, pl.ds(m * FPS, FPS)])


@functools.cache
def _agg_kernel():
    mesh = plsc.VectorSubcoreMesh(
        core_axis_name="c", subcore_axis_name="s",
        num_cores=NC, num_subcores=NS)
    return pl.kernel(
        _agg_kernel_body,
        out_type=jax.ShapeDtypeStruct((2, O, NH), jnp.float32),
        mesh=mesh,
        compiler_params=pltpu.CompilerParams(needs_layout_passes=False),
        scratch_types=[
            pltpu.VMEM((FPS, N), jnp.float32),
            pltpu.VMEM((FPS, NH), jnp.float32),
            pltpu.VMEM((ECH,), jnp.int32),
            pltpu.VMEM((ECH,), jnp.int32),
            pltpu.VMEM((ECH,), jnp.float32),
        ],
    )


def _epilogue_body(p_ref, zp_ref, s_ref, b_ref, o_ref):
    p = p_ref[...]
    z = jnp.sum(zp_ref[...], axis=1, keepdims=True) + jnp.float32(N)
    t = (p + s_ref[...]) / z
    t = jnp.maximum(t, ALPHA * t)
    nrm = jnp.maximum(jnp.sqrt(jnp.sum(t * t, axis=1, keepdims=True)),
                      jnp.float32(1e-12))
    o_ref[...] = t / nrm + b_ref[...]


@functools.cache
def _epilogue():
    return pl.pallas_call(
        _epilogue_body,
        grid=(N // BROWS,),
        in_specs=[
            pl.BlockSpec((BROWS, O), lambda i: (i, 0)),
            pl.BlockSpec((BROWS, NS), lambda i: (i, 0)),
            pl.BlockSpec((1, O), lambda i: (0, 0)),
            pl.BlockSpec((1, O), lambda i: (0, 0)),
        ],
        out_specs=pl.BlockSpec((BROWS, O), lambda i: (i, 0)),
        out_shape=jax.ShapeDtypeStruct((N, O), jnp.float32),
    )


def kernel(x, adj, weight, a, bias):
    a1 = a[:O, 0].reshape(1, O)
    a2 = a[O:, 0].reshape(1, O)
    h, wh1, wh2, s = _prologue()(x, weight, a1, a2)
    src = adj[0]
    dst = adj[1]
    w, zpart = _edge_kernel()(src, dst, wh1.reshape(N), wh2.reshape(N))
    ht = h.T
    pt = _agg_kernel()(ht, src, dst, w)
    p = jnp.concatenate([pt[0].T, pt[1].T], axis=0)
    return _epilogue()(p, zpart.T, s, bias.reshape(1, O))
